# baseline (speedup 1.0000x reference)
import functools

import jax
import jax.numpy as jnp
from jax import lax
from jax.experimental import pallas as pl
from jax.experimental.pallas import tpu as pltpu
from jax.experimental.pallas import tpu_sc as plsc

VOCAB = 32000
EMBED_DIM = 128
HIDDEN = 256
N_DOMAINS = 5
BATCH = 4096
SEQ = 200

NC = 2
NS = 16
LANES = 16
WORDS = EMBED_DIM // 2
CWORDS = WORDS // NC
NVEC = CWORDS // LANES
ROWS_PER_W = BATCH // NS
IDX_ROWS = 64
NPHASE = ROWS_PER_W // IDX_ROWS
UNROLL = 10

_mesh = plsc.VectorSubcoreMesh(core_axis_name="c", subcore_axis_name="s")


def _accum(buf, acc):
  mask_hi = jnp.full((LANES,), jnp.int32(-65536))

  def step(i, acc):
    acc = list(acc)
    for u in range(UNROLL):
      s = i * UNROLL + u
      for q in range(NVEC):
        w = buf[s, pl.ds(LANES * q, LANES)]
        even = lax.bitcast_convert_type(w << 16, jnp.float32)
        odd = lax.bitcast_convert_type(w & mask_hi, jnp.float32)
        acc[2 * q] = acc[2 * q] + even
        acc[2 * q + 1] = acc[2 * q + 1] + odd
    return tuple(acc)
  return lax.fori_loop(0, SEQ // UNROLL, step, acc)


@functools.partial(
    pl.kernel,
    out_type=jax.ShapeDtypeStruct((NC, BATCH, CWORDS), jnp.int32),
    mesh=_mesh,
    compiler_params=pltpu.CompilerParams(use_tc_tiling_on_sc=False),
    scratch_types=[
        pltpu.VMEM((IDX_ROWS, SEQ), jnp.int32),
        pltpu.VMEM((IDX_ROWS, SEQ), jnp.int32),
        pltpu.VMEM((SEQ, CWORDS), jnp.int32),
        pltpu.VMEM((SEQ, CWORDS), jnp.int32),
        pltpu.VMEM((SEQ, CWORDS), jnp.int32),
        pltpu.VMEM((SEQ, CWORDS), jnp.int32),
        pltpu.VMEM((IDX_ROWS, CWORDS), jnp.int32),
        pltpu.VMEM((IDX_ROWS, CWORDS), jnp.int32),
        pltpu.VMEM_SHARED((VOCAB, CWORDS), jnp.int32),
        pltpu.SemaphoreType.DMA,
        pltpu.SemaphoreType.DMA,
        pltpu.SemaphoreType.DMA,
        pltpu.SemaphoreType.DMA,
        pltpu.SemaphoreType.DMA,
        pltpu.SemaphoreType.DMA,
        pltpu.SemaphoreType.DMA,
    ],
)
def _pool_kernel(ids_hbm, table_hbm, out_hbm,
                 idx_a, idx_b, g0, g1, g2, g3, out_a, out_b, tab_s,
                 s0, s1, s2, s3, s_idx, so0, so1):
  core = lax.axis_index("c")
  sub = lax.axis_index("s")

  slab = VOCAB // NS
  pltpu.sync_copy(
      table_hbm.at[pl.ds(sub * slab, slab), pl.ds(core * CWORDS, CWORDS)],
      tab_s.at[pl.ds(sub * slab, slab)])
  plsc.subcore_barrier()

  bufs = (g0, g1, g2, g3)
  sems = (s0, s1, s2, s3)
  obufs = (out_a, out_b)
  osems = (so0, so1)

  def issue(idx_v, r, buf, sem):
    pltpu.async_copy(tab_s.at[idx_v.at[r, pl.ds(0, SEQ)]], buf, sem)

  def wait(idx_v, buf, sem):
    pltpu.make_async_copy(
        tab_s.at[idx_v.at[0, pl.ds(0, SEQ)]], buf, sem).wait()

  def ids_src(phase):
    return ids_hbm.at[
        pl.ds(sub * ROWS_PER_W + phase * IDX_ROWS, IDX_ROWS)]

  def out_dst(phase):
    return out_hbm.at[core, pl.ds(sub * ROWS_PER_W + phase * IDX_ROWS,
                                  IDX_ROWS)]

  zeros = tuple(jnp.zeros((LANES,), jnp.float32) for _ in range(2 * NVEC))
  inv_s = jnp.float32(1.0 / SEQ)
  mask_hi = jnp.full((LANES,), jnp.int32(-65536))

  pltpu.sync_copy(ids_src(0), idx_a)

  for phase in range(NPHASE):
    idx_v = idx_a if phase % 2 == 0 else idx_b
    idx_n = idx_b if phase % 2 == 0 else idx_a
    out_v = obufs[phase % 2]
    if phase > 0:
      pltpu.make_async_copy(ids_src(phase), idx_v, s_idx).wait()
    if phase + 1 < NPHASE:
      pltpu.async_copy(ids_src(phase + 1), idx_n, s_idx)
    if phase >= 2:
      pltpu.make_async_copy(out_v, out_dst(phase - 2),
                            osems[phase % 2]).wait()

    for k in range(4):
      issue(idx_v, k, bufs[k], sems[k])

    def body(i, carry):
      for k in range(4):
        row = 4 * i + k
        b = bufs[k]
        sm = sems[k]
        wait(idx_v, b, sm)
        acc = _accum(b, zeros)
        nxt = row + 4

        @pl.when(nxt < IDX_ROWS)
        def _():
          issue(idx_v, nxt, b, sm)

        for q in range(NVEC):
          e = lax.bitcast_convert_type(acc[2 * q] * inv_s, jnp.int32)
          o = lax.bitcast_convert_type(acc[2 * q + 1] * inv_s, jnp.int32)
          w = (o & mask_hi) | lax.shift_right_logical(e, 16)
          out_v[row, pl.ds(LANES * q, LANES)] = w
      return carry

    lax.fori_loop(0, IDX_ROWS // 4, body, jnp.int32(0))

    pltpu.async_copy(out_v, out_dst(phase), osems[phase % 2])

  pltpu.make_async_copy(obufs[(NPHASE - 2) % 2], out_dst(NPHASE - 2),
                        osems[(NPHASE - 2) % 2]).wait()
  pltpu.make_async_copy(obufs[(NPHASE - 1) % 2], out_dst(NPHASE - 1),
                        osems[(NPHASE - 1) % 2]).wait()


def _mlp_body(x_ref, w1_ref, b1_ref, w2_ref, b2_ref, o_ref):
  x = jnp.concatenate([x_ref[0], x_ref[1]], axis=1).astype(jnp.float32)
  h = jnp.dot(x, w1_ref[...], preferred_element_type=jnp.float32)
  h = jnp.maximum(h + b1_ref[...], 0.0)
  o_ref[...] = (
      jnp.dot(h, w2_ref[...], preferred_element_type=jnp.float32) + b2_ref[...]
  )


def _mlp(halves, W1, b1, W2p, b2p):
  return pl.pallas_call(
      _mlp_body,
      out_shape=jax.ShapeDtypeStruct((BATCH, 128), jnp.float32),
  )(halves, W1, b1, W2p, b2p)


def kernel(input_ids, emb_table, W1, b1, W2, b2):
  ids = input_ids.astype(jnp.int32)
  table_words = lax.bitcast_convert_type(
      emb_table.astype(jnp.bfloat16).reshape(VOCAB, WORDS, 2), jnp.int32)
  packed = _pool_kernel(ids, table_words)
  halves = lax.bitcast_convert_type(packed, jnp.bfloat16).reshape(
      NC, BATCH, EMBED_DIM // 2)
  W2p = jnp.pad(W2, ((0, 0), (0, 128 - N_DOMAINS)))
  b2p = jnp.pad(b2, (0, 128 - N_DOMAINS)).reshape(1, 128)
  logits = _mlp(halves, W1, b1.reshape(1, HIDDEN), W2p, b2p)
  return logits[:, :N_DOMAINS]


# speedup vs baseline: 1.0486x; 1.0486x over previous
import functools

import jax
import jax.numpy as jnp
import numpy as np
from jax import lax
from jax.experimental import pallas as pl
from jax.experimental.pallas import tpu as pltpu
from jax.experimental.pallas import tpu_sc as plsc

VOCAB = 32000
EMBED_DIM = 128
HIDDEN = 256
N_DOMAINS = 5
BATCH = 4096
SEQ = 200

NC = 2
NS = 16
LANES = 16
WORDS = EMBED_DIM // 2
CWORDS = WORDS // NC
NVEC = CWORDS // LANES
ROWS_PER_W = BATCH // NS
IDX_ROWS = 64
NPHASE = ROWS_PER_W // IDX_ROWS
UNROLL = 10

_mesh = plsc.VectorSubcoreMesh(core_axis_name="c", subcore_axis_name="s")

_PERM = np.concatenate(
    [np.concatenate([np.arange(32 * q, 32 * q + 32, 2),
                     np.arange(32 * q + 1, 32 * q + 32, 2)])
     for q in range(WORDS // LANES)])


def _accum(buf, acc):
  mask_hi = jnp.full((LANES,), jnp.int32(-65536))

  def step(i, acc):
    acc = list(acc)
    for u in range(UNROLL):
      s = i * UNROLL + u
      for q in range(NVEC):
        w = buf[s, pl.ds(LANES * q, LANES)]
        even = lax.bitcast_convert_type(w << 16, jnp.float32)
        odd = lax.bitcast_convert_type(w & mask_hi, jnp.float32)
        acc[2 * q] = acc[2 * q] + even
        acc[2 * q + 1] = acc[2 * q + 1] + odd
    return tuple(acc)
  return lax.fori_loop(0, SEQ // UNROLL, step, acc)


@functools.partial(
    pl.kernel,
    out_type=jax.ShapeDtypeStruct((NC, BATCH, EMBED_DIM // 2), jnp.float32),
    mesh=_mesh,
    compiler_params=pltpu.CompilerParams(use_tc_tiling_on_sc=False),
    scratch_types=[
        pltpu.VMEM((IDX_ROWS, SEQ), jnp.int32),
        pltpu.VMEM((IDX_ROWS, SEQ), jnp.int32),
        pltpu.VMEM((SEQ, CWORDS), jnp.int32),
        pltpu.VMEM((SEQ, CWORDS), jnp.int32),
        pltpu.VMEM((SEQ, CWORDS), jnp.int32),
        pltpu.VMEM((SEQ, CWORDS), jnp.int32),
        pltpu.VMEM((IDX_ROWS, EMBED_DIM // 2), jnp.float32),
        pltpu.VMEM((IDX_ROWS, EMBED_DIM // 2), jnp.float32),
        pltpu.VMEM_SHARED((VOCAB, CWORDS), jnp.int32),
        pltpu.SemaphoreType.DMA,
        pltpu.SemaphoreType.DMA,
        pltpu.SemaphoreType.DMA,
        pltpu.SemaphoreType.DMA,
        pltpu.SemaphoreType.DMA,
        pltpu.SemaphoreType.DMA,
        pltpu.SemaphoreType.DMA,
    ],
)
def _pool_kernel(ids_hbm, table_hbm, out_hbm,
                 idx_a, idx_b, g0, g1, g2, g3, out_a, out_b, tab_s,
                 s0, s1, s2, s3, s_idx, so0, so1):
  core = lax.axis_index("c")
  sub = lax.axis_index("s")

  slab = VOCAB // NS
  pltpu.sync_copy(
      table_hbm.at[pl.ds(sub * slab, slab), pl.ds(core * CWORDS, CWORDS)],
      tab_s.at[pl.ds(sub * slab, slab)])
  plsc.subcore_barrier()

  bufs = (g0, g1, g2, g3)
  sems = (s0, s1, s2, s3)
  obufs = (out_a, out_b)
  osems = (so0, so1)

  def issue(idx_v, r, buf, sem):
    pltpu.async_copy(tab_s.at[idx_v.at[r, pl.ds(0, SEQ)]], buf, sem)

  def wait(idx_v, buf, sem):
    pltpu.make_async_copy(
        tab_s.at[idx_v.at[0, pl.ds(0, SEQ)]], buf, sem).wait()

  def ids_src(phase):
    return ids_hbm.at[
        pl.ds(sub * ROWS_PER_W + phase * IDX_ROWS, IDX_ROWS)]

  def out_dst(phase):
    return out_hbm.at[core, pl.ds(sub * ROWS_PER_W + phase * IDX_ROWS,
                                  IDX_ROWS)]

  zeros = tuple(jnp.zeros((LANES,), jnp.float32) for _ in range(2 * NVEC))
  inv_s = jnp.float32(1.0 / SEQ)

  pltpu.sync_copy(ids_src(0), idx_a)

  for phase in range(NPHASE):
    idx_v = idx_a if phase % 2 == 0 else idx_b
    idx_n = idx_b if phase % 2 == 0 else idx_a
    out_v = obufs[phase % 2]
    if phase > 0:
      pltpu.make_async_copy(ids_src(phase), idx_v, s_idx).wait()
    if phase + 1 < NPHASE:
      pltpu.async_copy(ids_src(phase + 1), idx_n, s_idx)
    if phase >= 2:
      pltpu.make_async_copy(out_v, out_dst(phase - 2),
                            osems[phase % 2]).wait()

    for k in range(4):
      issue(idx_v, k, bufs[k], sems[k])

    def body(i, carry):
      for k in range(4):
        row = 4 * i + k
        b = bufs[k]
        sm = sems[k]
        wait(idx_v, b, sm)
        acc = _accum(b, zeros)
        nxt = row + 4

        @pl.when(nxt < IDX_ROWS)
        def _():
          issue(idx_v, nxt, b, sm)

        for j in range(2 * NVEC):
          out_v[row, pl.ds(LANES * j, LANES)] = acc[j] * inv_s
      return carry

    lax.fori_loop(0, IDX_ROWS // 4, body, jnp.int32(0))

    pltpu.async_copy(out_v, out_dst(phase), osems[phase % 2])

  pltpu.make_async_copy(obufs[(NPHASE - 2) % 2], out_dst(NPHASE - 2),
                        osems[(NPHASE - 2) % 2]).wait()
  pltpu.make_async_copy(obufs[(NPHASE - 1) % 2], out_dst(NPHASE - 1),
                        osems[(NPHASE - 1) % 2]).wait()


def _mlp_body(x_ref, w1_ref, b1_ref, w2_ref, b2_ref, o_ref):
  x = jnp.concatenate([x_ref[0], x_ref[1]], axis=1)
  h = jnp.dot(x, w1_ref[...], preferred_element_type=jnp.float32)
  h = jnp.maximum(h + b1_ref[...], 0.0)
  o_ref[...] = (
      jnp.dot(h, w2_ref[...], preferred_element_type=jnp.float32) + b2_ref[...]
  )


def _mlp(halves, W1, b1, W2p, b2p):
  return pl.pallas_call(
      _mlp_body,
      out_shape=jax.ShapeDtypeStruct((BATCH, 128), jnp.float32),
  )(halves, W1, b1, W2p, b2p)


def kernel(input_ids, emb_table, W1, b1, W2, b2):
  ids = input_ids.astype(jnp.int32)
  table_words = lax.bitcast_convert_type(
      emb_table.astype(jnp.bfloat16).reshape(VOCAB, WORDS, 2), jnp.int32)
  halves = _pool_kernel(ids, table_words)
  W1p = W1[_PERM, :]
  W2p = jnp.pad(W2, ((0, 0), (0, 128 - N_DOMAINS)))
  b2p = jnp.pad(b2, (0, 128 - N_DOMAINS)).reshape(1, 128)
  logits = _mlp(halves, W1p, b1.reshape(1, HIDDEN), W2p, b2p)
  return logits[:, :N_DOMAINS]
